# baseline (speedup 1.0000x reference)
import functools

import jax
import jax.numpy as jnp
from jax import lax
from jax.experimental import pallas as pl
from jax.experimental.pallas import tpu as pltpu
from jax.experimental.pallas import tpu_sc as plsc

NC = 2
NS = 16
NW = NC * NS

CHUNK = 128
K = 8


@functools.partial(jax.jit, static_argnums=(2, 3))
def _sc_gather(idx_flat, weight, n_rows, d):
    per_w = n_rows // NW
    n_chunks = per_w // CHUNK
    n_grp = n_chunks // K
    mesh = plsc.VectorSubcoreMesh(
        core_axis_name="c", subcore_axis_name="s", num_cores=NC, num_subcores=NS
    )

    @functools.partial(
        pl.kernel,
        out_type=jax.ShapeDtypeStruct((n_rows, d), jnp.float32),
        mesh=mesh,
        compiler_params=pltpu.CompilerParams(use_tc_tiling_on_sc=False),
        scratch_types=[
            pltpu.VMEM((per_w,), jnp.int32),
            pltpu.VMEM((K, CHUNK, d), jnp.float32),
            pltpu.SemaphoreType.DMA((K,)),
        ],
    )
    def k(idx_hbm, table_hbm, out_hbm, idx_v, bufs, gsem):
        wid = lax.axis_index("s") * NC + lax.axis_index("c")
        base = wid * per_w
        pltpu.sync_copy(idx_hbm.at[pl.ds(base, per_w)], idx_v)

        def gather(j, b):
            pltpu.async_copy(
                table_hbm.at[idx_v.at[pl.ds(j * CHUNK, CHUNK)]],
                bufs.at[b],
                gsem.at[b],
            )

        for b in range(K):
            gather(b, b)

        def grp(g, carry):
            for b in range(K):
                j = g * K + b
                pltpu.make_async_copy(
                    table_hbm.at[idx_v.at[pl.ds(0, CHUNK)]], bufs.at[b], gsem.at[b]
                ).wait()
                pltpu.sync_copy(bufs.at[b], out_hbm.at[pl.ds(base + j * CHUNK, CHUNK)])
                gather(jnp.minimum(j + K, n_chunks - 1), b)
            return carry

        lax.fori_loop(0, n_grp, grp, 0)
        for b in range(K):
            pltpu.make_async_copy(
                table_hbm.at[idx_v.at[pl.ds(0, CHUNK)]], bufs.at[b], gsem.at[b]
            ).wait()

    return k(idx_flat, weight)


def kernel(inputs, weight):
    b, s = inputs.shape
    v, d = weight.shape
    idx_flat = inputs.reshape(-1).astype(jnp.int32)
    out = _sc_gather(idx_flat, weight, b * s, d)
    return out.reshape(b, s, d)


# speedup vs baseline: 1.2201x; 1.2201x over previous
import functools

import jax
import jax.numpy as jnp
from jax import lax
from jax.experimental import pallas as pl
from jax.experimental.pallas import tpu as pltpu
from jax.experimental.pallas import tpu_sc as plsc

NC = 2
NS = 16
NW = NC * NS

CHUNK = 128
K = 4


@functools.partial(jax.jit, static_argnums=(2,))
def _sc_gather(idx_flat, table, n_rows):
    d = table.shape[1]
    per_w = n_rows // NW
    n_chunks = per_w // CHUNK
    n_grp = n_chunks // K
    mesh = plsc.VectorSubcoreMesh(
        core_axis_name="c", subcore_axis_name="s", num_cores=NC, num_subcores=NS
    )

    @functools.partial(
        pl.kernel,
        out_type=jax.ShapeDtypeStruct((n_rows, d), jnp.float32),
        mesh=mesh,
        compiler_params=pltpu.CompilerParams(use_tc_tiling_on_sc=False),
        scratch_types=[
            pltpu.VMEM((per_w,), jnp.int32),
            pltpu.VMEM((K, CHUNK, d), jnp.float32),
            pltpu.SemaphoreType.DMA((K,)),
        ],
    )
    def k(idx_hbm, table_hbm, out_hbm, idx_v, bufs, gsem):
        wid = lax.axis_index("s") * NC + lax.axis_index("c")
        base = wid * per_w
        pltpu.sync_copy(idx_hbm.at[pl.ds(base, per_w)], idx_v)

        def gather(j, b):
            pltpu.async_copy(
                table_hbm.at[idx_v.at[pl.ds(j * CHUNK, CHUNK)]],
                bufs.at[b],
                gsem.at[b],
            )

        for b in range(K):
            gather(b, b)

        def grp(g, carry):
            for b in range(K):
                j = g * K + b
                pltpu.make_async_copy(
                    table_hbm.at[idx_v.at[pl.ds(0, CHUNK)]], bufs.at[b], gsem.at[b]
                ).wait()
                pltpu.sync_copy(bufs.at[b], out_hbm.at[pl.ds(base + j * CHUNK, CHUNK)])
                gather(jnp.minimum(j + K, n_chunks - 1), b)
            return carry

        lax.fori_loop(0, n_grp, grp, 0)
        for b in range(K):
            pltpu.make_async_copy(
                table_hbm.at[idx_v.at[pl.ds(0, CHUNK)]], bufs.at[b], gsem.at[b]
            ).wait()

    return k(idx_flat, table)


def kernel(inputs, weight):
    b, s = inputs.shape
    v, d = weight.shape
    idx_flat = inputs.reshape(-1).astype(jnp.int32)
    table = jnp.pad(weight, ((0, 0), (0, 128 - d)))
    out = _sc_gather(idx_flat, table, b * s)
    return out[:, :d].reshape(b, s, d)
